# baseline (speedup 1.0000x reference)
import functools

import jax
import jax.numpy as jnp
from jax import lax
from jax.experimental import pallas as pl
from jax.experimental.pallas import tpu as pltpu
from jax.experimental.pallas import tpu_sc as plsc

ENTITY_VOCAB = 1000000
EMBED_DIM = 64
BATCH = 4096
HIST = 50

_INFO = plsc.get_sparse_core_info()
_NC = _INFO.num_cores
_NS = _INFO.num_subcores
_NW = _NC * _NS
_NBLK = ENTITY_VOCAB // 128
_NBUF_A = 4
_NBUF_B = 3
_BPW = BATCH // _NW


def _iota16():
    return lax.iota(jnp.int32, 16)


def _wid():
    return lax.axis_index("s") * _NC + lax.axis_index("c")


def _permute_block(tile_ref, pair_ref, rows):
    def yloop(y, carry):
        for m in range(8):
            col = jnp.full((16,), 2 * y + m // 4, jnp.int32)
            v = plsc.load_gather(tile_ref, [rows[m % 4], col])
            pair_ref[y, pl.ds(16 * m, 16)] = v
        return carry

    lax.fori_loop(0, 64, yloop, 0)


def _transpose_body(tabT, TT, tile_v, pair_v, tail_in, tail_out, sem_in,
                    sem_out):
    w = _wid()
    base = 244 * w + jnp.minimum(w, 5)
    cnt = jnp.where(w < 5, 245, 244)
    cnt = jnp.where(w == _NW - 1, cnt - 1, cnt)
    jend = base + cnt
    rows = [_iota16() + 16 * q for q in range(4)]

    def g_in(j, b):
        return pltpu.make_async_copy(
            tabT.at[:, pl.ds(128 * j, 128)], tile_v.at[b], sem_in.at[b])

    def g_out(j, b):
        return pltpu.make_async_copy(
            pair_v.at[b], TT.at[pl.ds(64 * j, 64), :], sem_out.at[b])

    for b in range(_NBUF_A):
        g_in(base + b, b).start()

    def step(t, carry):
        for b in range(_NBUF_A):
            j = base + _NBUF_A * t + b

            @pl.when(j < jend)
            def _():
                g_in(j, b).wait()
                _permute_block(tile_v.at[b], pair_v.at[b], rows)
                g_out(j, b).start()
                g_out(j, b).wait()

                @pl.when(j + _NBUF_A < jend)
                def _():
                    g_in(j + _NBUF_A, b).start()

        return carry

    lax.fori_loop(0, (245 + _NBUF_A - 1) // _NBUF_A, step, 0)

    @pl.when(w == _NW - 1)
    def _():
        pltpu.sync_copy(tabT.at[:, pl.ds(128 * _NBLK, 64)], tail_in)

        def yloop(y, carry):
            for m in range(8):
                col = jnp.full((16,), 2 * y + m // 4, jnp.int32)
                v = plsc.load_gather(tail_in, [rows[m % 4], col])
                tail_out[y, pl.ds(16 * m, 16)] = v
            return carry

        lax.fori_loop(0, 32, yloop, 0)
        pltpu.sync_copy(tail_out, TT.at[pl.ds(64 * _NBLK, 32), :])


@jax.jit
def _phase_a(tabT):
    mesh = plsc.VectorSubcoreMesh(core_axis_name="c", subcore_axis_name="s")
    fn = pl.kernel(
        _transpose_body,
        mesh=mesh,
        out_type=jax.ShapeDtypeStruct((ENTITY_VOCAB // 2, 128), jnp.float32),
        scratch_types=[
            pltpu.VMEM((_NBUF_A, 64, 128), jnp.float32),
            pltpu.VMEM((_NBUF_A, 64, 128), jnp.float32),
            pltpu.VMEM((64, 64), jnp.float32),
            pltpu.VMEM((32, 128), jnp.float32),
            pltpu.SemaphoreType.DMA((_NBUF_A,)),
            pltpu.SemaphoreType.DMA((_NBUF_A,)),
        ],
        compiler_params=pltpu.CompilerParams(needs_layout_passes=False),
    )
    return fn(tabT)


def _gather_body(TT, idxT, outT, idx_v, p_v, half_v, rows_v, otile_v,
                 sem_rows, sem_out):
    w = _wid()
    pltpu.sync_copy(idxT.at[:, pl.ds(128 * w, 128)], idx_v)

    def prep(t, carry):
        for m in range(8):
            r = idx_v[t, pl.ds(16 * m, 16)]
            p_v[t, pl.ds(16 * m, 16)] = lax.shift_right_logical(r, 1)
            half_v[t, pl.ds(16 * m, 16)] = lax.shift_left(
                lax.bitwise_and(r, 1), 6)
        return carry

    lax.fori_loop(0, HIST, prep, 0)

    lanes = [_iota16() + 16 * m for m in range(8)]

    def g_rows(h, b):
        return pltpu.make_async_copy(TT.at[p_v.at[h]], rows_v.at[b],
                                     sem_rows.at[b])

    def g_out(h, b):
        return pltpu.make_async_copy(
            otile_v.at[b], outT.at[h].at[:, pl.ds(128 * w, 128)],
            sem_out.at[b])

    for b in range(_NBUF_B):
        g_rows(b, b).start()

    def step(t, carry):
        for b in range(_NBUF_B):
            h = _NBUF_B * t + b

            @pl.when(h < HIST)
            def _():
                g_rows(h, b).wait()
                cols = [half_v[h, pl.ds(16 * m, 16)] for m in range(8)]

                def cloop(c, carry2):
                    for m in range(8):
                        v = plsc.load_gather(rows_v.at[b],
                                             [lanes[m], cols[m] + c])
                        otile_v[b, c, pl.ds(16 * m, 16)] = v
                    return carry2

                lax.fori_loop(0, 64, cloop, 0)
                g_out(h, b).start()
                g_out(h, b).wait()

                @pl.when(h + _NBUF_B < HIST)
                def _():
                    g_rows(h + _NBUF_B, b).start()

        return carry

    lax.fori_loop(0, (HIST + _NBUF_B - 1) // _NBUF_B, step, 0)


@jax.jit
def _phase_b(TT, idxT):
    mesh = plsc.VectorSubcoreMesh(core_axis_name="c", subcore_axis_name="s")
    fn = pl.kernel(
        _gather_body,
        mesh=mesh,
        out_type=jax.ShapeDtypeStruct((HIST, EMBED_DIM, BATCH), jnp.float32),
        scratch_types=[
            pltpu.VMEM((HIST, 128), jnp.int32),
            pltpu.VMEM((HIST, 128), jnp.int32),
            pltpu.VMEM((HIST, 128), jnp.int32),
            pltpu.VMEM((_NBUF_B, 128, 128), jnp.float32),
            pltpu.VMEM((_NBUF_B, EMBED_DIM, 128), jnp.float32),
            pltpu.SemaphoreType.DMA((_NBUF_B,)),
            pltpu.SemaphoreType.DMA((_NBUF_B,)),
        ],
        compiler_params=pltpu.CompilerParams(needs_layout_passes=False),
    )
    return fn(TT, idxT)


def kernel(entities, entity_table):
    tabT = entity_table.T
    TT = _phase_a(tabT)
    idxT = entities.T
    outT = _phase_b(TT, idxT)
    return jnp.transpose(outT, (2, 0, 1))


# speedup vs baseline: 1.0524x; 1.0524x over previous
import functools

import jax
import jax.numpy as jnp
from jax import lax
from jax.experimental import pallas as pl
from jax.experimental.pallas import tpu as pltpu
from jax.experimental.pallas import tpu_sc as plsc

ENTITY_VOCAB = 1000000
EMBED_DIM = 64
BATCH = 4096
HIST = 50

_INFO = plsc.get_sparse_core_info()
_NC = _INFO.num_cores
_NS = _INFO.num_subcores
_NW = _NC * _NS
_NBLK = ENTITY_VOCAB // 128
_NBUF_A = 4
_NBUF_B = 3
_BPW = BATCH // _NW


def _iota16():
    return lax.iota(jnp.int32, 16)


def _wid():
    return lax.axis_index("s") * _NC + lax.axis_index("c")


def _permute_rows(tile_ref, pair_ref, rows, nrows):
    def yloop(yy, carry):
        for dy in range(4):
            y = 4 * yy + dy
            c0 = jnp.full((16,), 2 * y, jnp.int32)
            c1 = c0 + 1
            for m in range(8):
                v = plsc.load_gather(tile_ref,
                                     [rows[m % 4], c0 if m < 4 else c1])
                pair_ref[y, pl.ds(16 * m, 16)] = v
        return carry

    lax.fori_loop(0, nrows // 4, yloop, 0)


def _transpose_body(tabT, TT, tile_v, pair_v, tail_in, tail_out, sem_in,
                    sem_out):
    w = _wid()
    base = 244 * w + jnp.minimum(w, 5)
    cnt = jnp.where(w < 5, 245, 244)
    cnt = jnp.where(w == _NW - 1, cnt - 1, cnt)
    jend = base + cnt
    rows = [_iota16() + 16 * q for q in range(4)]

    def g_in(j, b):
        return pltpu.make_async_copy(
            tabT.at[:, pl.ds(128 * j, 128)], tile_v.at[b], sem_in.at[b])

    def g_out(j, b):
        return pltpu.make_async_copy(
            pair_v.at[b], TT.at[pl.ds(64 * j, 64), :], sem_out.at[b])

    for b in range(_NBUF_A):
        g_in(base + b, b).start()

    def step(t, carry):
        for b in range(_NBUF_A):
            j = base + _NBUF_A * t + b

            @pl.when(j < jend)
            def _():
                g_in(j, b).wait()

                @pl.when(t > 0)
                def _():
                    g_out(j - _NBUF_A, b).wait()

                _permute_rows(tile_v.at[b], pair_v.at[b], rows, 64)
                g_out(j, b).start()

                @pl.when(j + _NBUF_A < jend)
                def _():
                    g_in(j + _NBUF_A, b).start()

        return carry

    lax.fori_loop(0, (245 + _NBUF_A - 1) // _NBUF_A, step, 0)

    for b in range(_NBUF_A):
        jlast = jend - 1 - lax.rem(jend - 1 - base - b, _NBUF_A)
        g_out(jlast, b).wait()

    @pl.when(w == _NW - 1)
    def _():
        pltpu.sync_copy(tabT.at[:, pl.ds(128 * _NBLK, 64)], tail_in)
        _permute_rows(tail_in, tail_out, rows, 32)
        pltpu.sync_copy(tail_out, TT.at[pl.ds(64 * _NBLK, 32), :])


@jax.jit
def _phase_a(tabT):
    mesh = plsc.VectorSubcoreMesh(core_axis_name="c", subcore_axis_name="s")
    fn = pl.kernel(
        _transpose_body,
        mesh=mesh,
        out_type=jax.ShapeDtypeStruct((ENTITY_VOCAB // 2, 128), jnp.float32),
        scratch_types=[
            pltpu.VMEM((_NBUF_A, 64, 128), jnp.float32),
            pltpu.VMEM((_NBUF_A, 64, 128), jnp.float32),
            pltpu.VMEM((64, 64), jnp.float32),
            pltpu.VMEM((32, 128), jnp.float32),
            pltpu.SemaphoreType.DMA((_NBUF_A,)),
            pltpu.SemaphoreType.DMA((_NBUF_A,)),
        ],
        compiler_params=pltpu.CompilerParams(needs_layout_passes=False),
    )
    return fn(tabT)


def _gather_body(TT, idxT, outT, idx_v, p_v, half_v, rows_v, otile_v,
                 sem_rows, sem_out):
    w = _wid()
    pltpu.sync_copy(idxT.at[:, pl.ds(128 * w, 128)], idx_v)

    def prep(t, carry):
        for m in range(8):
            r = idx_v[t, pl.ds(16 * m, 16)]
            p_v[t, pl.ds(16 * m, 16)] = lax.shift_right_logical(r, 1)
            half_v[t, pl.ds(16 * m, 16)] = lax.shift_left(
                lax.bitwise_and(r, 1), 6)
        return carry

    lax.fori_loop(0, HIST, prep, 0)

    lanes = [_iota16() + 16 * m for m in range(8)]

    def g_rows(h, b):
        return pltpu.make_async_copy(TT.at[p_v.at[h]], rows_v.at[b],
                                     sem_rows.at[b])

    def g_out(h, b):
        return pltpu.make_async_copy(
            otile_v.at[b], outT.at[h].at[:, pl.ds(128 * w, 128)],
            sem_out.at[b])

    for b in range(_NBUF_B):
        g_rows(b, b).start()

    def step(t, carry):
        for b in range(_NBUF_B):
            h = _NBUF_B * t + b

            @pl.when(h < HIST)
            def _():
                g_rows(h, b).wait()

                @pl.when(t > 0)
                def _():
                    g_out(h - _NBUF_B, b).wait()

                cols = [half_v[h, pl.ds(16 * m, 16)] for m in range(8)]

                def cloop(cc, carry2):
                    for dc in range(4):
                        c = 4 * cc + dc
                        for m in range(8):
                            v = plsc.load_gather(rows_v.at[b],
                                                 [lanes[m], cols[m] + c])
                            otile_v[b, c, pl.ds(16 * m, 16)] = v
                    return carry2

                lax.fori_loop(0, 16, cloop, 0)
                g_out(h, b).start()

                @pl.when(h + _NBUF_B < HIST)
                def _():
                    g_rows(h + _NBUF_B, b).start()

        return carry

    lax.fori_loop(0, (HIST + _NBUF_B - 1) // _NBUF_B, step, 0)

    for b in range(_NBUF_B):
        hlast = HIST - 1 - lax.rem(jnp.int32(HIST - 1 - b), _NBUF_B)
        g_out(hlast, b).wait()


@jax.jit
def _phase_b(TT, idxT):
    mesh = plsc.VectorSubcoreMesh(core_axis_name="c", subcore_axis_name="s")
    fn = pl.kernel(
        _gather_body,
        mesh=mesh,
        out_type=jax.ShapeDtypeStruct((HIST, EMBED_DIM, BATCH), jnp.float32),
        scratch_types=[
            pltpu.VMEM((HIST, 128), jnp.int32),
            pltpu.VMEM((HIST, 128), jnp.int32),
            pltpu.VMEM((HIST, 128), jnp.int32),
            pltpu.VMEM((_NBUF_B, 128, 128), jnp.float32),
            pltpu.VMEM((_NBUF_B, EMBED_DIM, 128), jnp.float32),
            pltpu.SemaphoreType.DMA((_NBUF_B,)),
            pltpu.SemaphoreType.DMA((_NBUF_B,)),
        ],
        compiler_params=pltpu.CompilerParams(needs_layout_passes=False),
    )
    return fn(TT, idxT)


def kernel(entities, entity_table):
    tabT = entity_table.T
    TT = _phase_a(tabT)
    idxT = entities.T
    outT = _phase_b(TT, idxT)
    return jnp.transpose(outT, (2, 0, 1))


# speedup vs baseline: 1.9668x; 1.8688x over previous
import functools

import jax
import jax.numpy as jnp
from jax import lax
from jax.experimental import pallas as pl
from jax.experimental.pallas import tpu as pltpu
from jax.experimental.pallas import tpu_sc as plsc

ENTITY_VOCAB = 1000000
EMBED_DIM = 64
BATCH = 4096
HIST = 50

_INFO = plsc.get_sparse_core_info()
_NC = _INFO.num_cores
_NS = _INFO.num_subcores
_NW = _NC * _NS
_NBLK = ENTITY_VOCAB // 128
_NBUF_A = 4
_NBUF_B = 3
_BPW = BATCH // _NW


def _iota16():
    return lax.iota(jnp.int32, 16)


def _wid():
    return lax.axis_index("s") * _NC + lax.axis_index("c")


def _permute_rows(tile_ref, pair_ref, rows, nrows):
    @plsc.parallel_loop(0, nrows, step=1, unroll=8)
    def yloop(y):
        c0 = jnp.full((16,), 2 * y, jnp.int32)
        c1 = c0 + 1
        for m in range(8):
            v = plsc.load_gather(tile_ref,
                                 [rows[m % 4], c0 if m < 4 else c1])
            pair_ref[y, pl.ds(16 * m, 16)] = v


def _transpose_body(tabT, TT, tile_v, pair_v, tail_in, tail_out, sem_in,
                    sem_out):
    w = _wid()
    base = 244 * w + jnp.minimum(w, 5)
    cnt = jnp.where(w < 5, 245, 244)
    cnt = jnp.where(w == _NW - 1, cnt - 1, cnt)
    jend = base + cnt
    rows = [_iota16() + 16 * q for q in range(4)]

    def g_in(j, b):
        return pltpu.make_async_copy(
            tabT.at[:, pl.ds(128 * j, 128)], tile_v.at[b], sem_in.at[b])

    def g_out(j, b):
        return pltpu.make_async_copy(
            pair_v.at[b], TT.at[pl.ds(64 * j, 64), :], sem_out.at[b])

    for b in range(_NBUF_A):
        g_in(base + b, b).start()

    def step(t, carry):
        for b in range(_NBUF_A):
            j = base + _NBUF_A * t + b

            @pl.when(j < jend)
            def _():
                g_in(j, b).wait()

                @pl.when(t > 0)
                def _():
                    g_out(j - _NBUF_A, b).wait()

                _permute_rows(tile_v.at[b], pair_v.at[b], rows, 64)
                g_out(j, b).start()

                @pl.when(j + _NBUF_A < jend)
                def _():
                    g_in(j + _NBUF_A, b).start()

        return carry

    lax.fori_loop(0, (245 + _NBUF_A - 1) // _NBUF_A, step, 0)

    for b in range(_NBUF_A):
        jlast = jend - 1 - lax.rem(jend - 1 - base - b, _NBUF_A)
        g_out(jlast, b).wait()

    @pl.when(w == _NW - 1)
    def _():
        pltpu.sync_copy(tabT.at[:, pl.ds(128 * _NBLK, 64)], tail_in)
        _permute_rows(tail_in, tail_out, rows, 32)
        pltpu.sync_copy(tail_out, TT.at[pl.ds(64 * _NBLK, 32), :])


@jax.jit
def _phase_a(tabT):
    mesh = plsc.VectorSubcoreMesh(core_axis_name="c", subcore_axis_name="s")
    fn = pl.kernel(
        _transpose_body,
        mesh=mesh,
        out_type=jax.ShapeDtypeStruct((ENTITY_VOCAB // 2, 128), jnp.float32),
        scratch_types=[
            pltpu.VMEM((_NBUF_A, 64, 128), jnp.float32),
            pltpu.VMEM((_NBUF_A, 64, 128), jnp.float32),
            pltpu.VMEM((64, 64), jnp.float32),
            pltpu.VMEM((32, 128), jnp.float32),
            pltpu.SemaphoreType.DMA((_NBUF_A,)),
            pltpu.SemaphoreType.DMA((_NBUF_A,)),
        ],
        compiler_params=pltpu.CompilerParams(needs_layout_passes=False),
    )
    return fn(tabT)


def _gather_body(TT, idxT, outT, idx_v, p_v, half_v, rows_v, otile_v,
                 sem_rows, sem_out):
    w = _wid()
    pltpu.sync_copy(idxT.at[:, pl.ds(128 * w, 128)], idx_v)

    @plsc.parallel_loop(0, HIST, step=1, unroll=2)
    def prep(t):
        for m in range(8):
            r = idx_v[t, pl.ds(16 * m, 16)]
            p_v[t, pl.ds(16 * m, 16)] = lax.shift_right_logical(r, 1)
            half_v[t, pl.ds(16 * m, 16)] = lax.shift_left(
                lax.bitwise_and(r, 1), 6)

    lanes = [_iota16() + 16 * m for m in range(8)]

    def g_rows(h, b):
        return pltpu.make_async_copy(TT.at[p_v.at[h]], rows_v.at[b],
                                     sem_rows.at[b])

    def g_out(h, b):
        return pltpu.make_async_copy(
            otile_v.at[b], outT.at[h].at[:, pl.ds(128 * w, 128)],
            sem_out.at[b])

    for b in range(_NBUF_B):
        g_rows(b, b).start()

    def step(t, carry):
        for b in range(_NBUF_B):
            h = _NBUF_B * t + b

            @pl.when(h < HIST)
            def _():
                g_rows(h, b).wait()

                @pl.when(t > 0)
                def _():
                    g_out(h - _NBUF_B, b).wait()

                cols = [half_v[h, pl.ds(16 * m, 16)] for m in range(8)]

                @plsc.parallel_loop(0, EMBED_DIM, step=1, unroll=8)
                def cloop(c):
                    for m in range(8):
                        v = plsc.load_gather(rows_v.at[b],
                                             [lanes[m], cols[m] + c])
                        otile_v[b, c, pl.ds(16 * m, 16)] = v
                g_out(h, b).start()

                @pl.when(h + _NBUF_B < HIST)
                def _():
                    g_rows(h + _NBUF_B, b).start()

        return carry

    lax.fori_loop(0, (HIST + _NBUF_B - 1) // _NBUF_B, step, 0)

    for b in range(_NBUF_B):
        hlast = HIST - 1 - lax.rem(jnp.int32(HIST - 1 - b), _NBUF_B)
        g_out(hlast, b).wait()


@jax.jit
def _phase_b(TT, idxT):
    mesh = plsc.VectorSubcoreMesh(core_axis_name="c", subcore_axis_name="s")
    fn = pl.kernel(
        _gather_body,
        mesh=mesh,
        out_type=jax.ShapeDtypeStruct((HIST, EMBED_DIM, BATCH), jnp.float32),
        scratch_types=[
            pltpu.VMEM((HIST, 128), jnp.int32),
            pltpu.VMEM((HIST, 128), jnp.int32),
            pltpu.VMEM((HIST, 128), jnp.int32),
            pltpu.VMEM((_NBUF_B, 128, 128), jnp.float32),
            pltpu.VMEM((_NBUF_B, EMBED_DIM, 128), jnp.float32),
            pltpu.SemaphoreType.DMA((_NBUF_B,)),
            pltpu.SemaphoreType.DMA((_NBUF_B,)),
        ],
        compiler_params=pltpu.CompilerParams(needs_layout_passes=False),
    )
    return fn(TT, idxT)


def kernel(entities, entity_table):
    tabT = entity_table.T
    TT = _phase_a(tabT)
    idxT = entities.T
    outT = _phase_b(TT, idxT)
    return jnp.transpose(outT, (2, 0, 1))


# speedup vs baseline: 7.4400x; 3.7827x over previous
import functools

import jax
import jax.numpy as jnp
from jax import lax
from jax.experimental import pallas as pl
from jax.experimental.pallas import tpu as pltpu
from jax.experimental.pallas import tpu_sc as plsc

ENTITY_VOCAB = 1000000
EMBED_DIM = 64
BATCH = 4096
HIST = 50

_INFO = plsc.get_sparse_core_info()
_NC = _INFO.num_cores
_NS = _INFO.num_subcores
_NW = _NC * _NS
_NBLK = ENTITY_VOCAB // 128
_NBUF_A = 4
_NBUF_B = 3
_BPW = BATCH // _NW


def _iota16():
    return lax.iota(jnp.int32, 16)


def _wid():
    return lax.axis_index("s") * _NC + lax.axis_index("c")


def _permute_rows(tile_ref, pair_ref, nrows):
    yvecs = [_iota16() + 16 * t for t in range(nrows // 16)]
    two_i = _iota16() * 2

    @plsc.parallel_loop(0, 128, step=1, unroll=4)
    def kbody(k):
        rowv = jnp.full((16,), lax.bitwise_and(k, 63), jnp.int32)
        colb = two_i + lax.shift_right_logical(k, 6)
        for t, yv in enumerate(yvecs):
            v = plsc.load_gather(tile_ref, [rowv, colb + 32 * t])
            plsc.store_scatter(pair_ref,
                               [yv, lax.bitwise_and(yv + k, 127)], v)


def _transpose_body(tabT, TT, tile_v, pair_v, tail_in, tail_out, sem_in,
                    sem_out):
    w = _wid()
    base = 244 * w + jnp.minimum(w, 5)
    cnt = jnp.where(w < 5, 245, 244)
    cnt = jnp.where(w == _NW - 1, cnt - 1, cnt)
    jend = base + cnt

    def g_in(j, b):
        return pltpu.make_async_copy(
            tabT.at[:, pl.ds(128 * j, 128)], tile_v.at[b], sem_in.at[b])

    def g_out(j, b):
        return pltpu.make_async_copy(
            pair_v.at[b], TT.at[pl.ds(64 * j, 64), :], sem_out.at[b])

    for b in range(_NBUF_A):
        g_in(base + b, b).start()

    def step(t, carry):
        for b in range(_NBUF_A):
            j = base + _NBUF_A * t + b

            @pl.when(j < jend)
            def _():
                g_in(j, b).wait()

                @pl.when(t > 0)
                def _():
                    g_out(j - _NBUF_A, b).wait()

                _permute_rows(tile_v.at[b], pair_v.at[b], 64)
                g_out(j, b).start()

                @pl.when(j + _NBUF_A < jend)
                def _():
                    g_in(j + _NBUF_A, b).start()

        return carry

    lax.fori_loop(0, (245 + _NBUF_A - 1) // _NBUF_A, step, 0)

    for b in range(_NBUF_A):
        jlast = jend - 1 - lax.rem(jend - 1 - base - b, _NBUF_A)
        g_out(jlast, b).wait()

    @pl.when(w == _NW - 1)
    def _():
        pltpu.sync_copy(tabT.at[:, pl.ds(128 * _NBLK, 64)], tail_in)
        _permute_rows(tail_in, tail_out, 32)
        pltpu.sync_copy(tail_out, TT.at[pl.ds(64 * _NBLK, 32), :])


@jax.jit
def _phase_a(tabT):
    mesh = plsc.VectorSubcoreMesh(core_axis_name="c", subcore_axis_name="s")
    fn = pl.kernel(
        _transpose_body,
        mesh=mesh,
        out_type=jax.ShapeDtypeStruct((ENTITY_VOCAB // 2, 128), jnp.float32),
        scratch_types=[
            pltpu.VMEM((_NBUF_A, 64, 128), jnp.float32),
            pltpu.VMEM((_NBUF_A, 64, 128), jnp.float32),
            pltpu.VMEM((64, 64), jnp.float32),
            pltpu.VMEM((32, 128), jnp.float32),
            pltpu.SemaphoreType.DMA((_NBUF_A,)),
            pltpu.SemaphoreType.DMA((_NBUF_A,)),
        ],
        compiler_params=pltpu.CompilerParams(needs_layout_passes=False),
    )
    return fn(tabT)


def _gather_body(TT, idxT, outT, idx_v, p_v, half_v, rows_v, otile_v,
                 sem_rows, sem_out):
    w = _wid()
    pltpu.sync_copy(idxT.at[:, pl.ds(128 * w, 128)], idx_v)

    @plsc.parallel_loop(0, HIST, step=1, unroll=2)
    def prep(t):
        for m in range(8):
            r = idx_v[t, pl.ds(16 * m, 16)]
            p = lax.shift_right_logical(r, 1)
            p_v[t, pl.ds(16 * m, 16)] = p
            half_v[t, pl.ds(16 * m, 16)] = (
                lax.shift_left(lax.bitwise_and(r, 1), 6)
                + lax.bitwise_and(p, 63))

    lanes = [_iota16() + 16 * m for m in range(8)]

    def g_rows(h, b):
        return pltpu.make_async_copy(TT.at[p_v.at[h]], rows_v.at[b],
                                     sem_rows.at[b])

    def g_out(h, b):
        return pltpu.make_async_copy(
            otile_v.at[b], outT.at[h].at[:, pl.ds(128 * w, 128)],
            sem_out.at[b])

    for b in range(_NBUF_B):
        g_rows(b, b).start()

    def step(t, carry):
        for b in range(_NBUF_B):
            h = _NBUF_B * t + b

            @pl.when(h < HIST)
            def _():
                g_rows(h, b).wait()

                @pl.when(t > 0)
                def _():
                    g_out(h - _NBUF_B, b).wait()

                cols = [half_v[h, pl.ds(16 * m, 16)] for m in range(8)]

                @plsc.parallel_loop(0, EMBED_DIM, step=1, unroll=8)
                def cloop(c):
                    for m in range(8):
                        col = lax.bitwise_and(cols[m] + c, 127)
                        v = plsc.load_gather(rows_v.at[b], [lanes[m], col])
                        otile_v[b, c, pl.ds(16 * m, 16)] = v
                g_out(h, b).start()

                @pl.when(h + _NBUF_B < HIST)
                def _():
                    g_rows(h + _NBUF_B, b).start()

        return carry

    lax.fori_loop(0, (HIST + _NBUF_B - 1) // _NBUF_B, step, 0)

    for b in range(_NBUF_B):
        hlast = HIST - 1 - lax.rem(jnp.int32(HIST - 1 - b), _NBUF_B)
        g_out(hlast, b).wait()


@jax.jit
def _phase_b(TT, idxT):
    mesh = plsc.VectorSubcoreMesh(core_axis_name="c", subcore_axis_name="s")
    fn = pl.kernel(
        _gather_body,
        mesh=mesh,
        out_type=jax.ShapeDtypeStruct((HIST, EMBED_DIM, BATCH), jnp.float32),
        scratch_types=[
            pltpu.VMEM((HIST, 128), jnp.int32),
            pltpu.VMEM((HIST, 128), jnp.int32),
            pltpu.VMEM((HIST, 128), jnp.int32),
            pltpu.VMEM((_NBUF_B, 128, 128), jnp.float32),
            pltpu.VMEM((_NBUF_B, EMBED_DIM, 128), jnp.float32),
            pltpu.SemaphoreType.DMA((_NBUF_B,)),
            pltpu.SemaphoreType.DMA((_NBUF_B,)),
        ],
        compiler_params=pltpu.CompilerParams(needs_layout_passes=False),
    )
    return fn(TT, idxT)


def kernel(entities, entity_table):
    tabT = entity_table.T
    TT = _phase_a(tabT)
    idxT = entities.T
    outT = _phase_b(TT, idxT)
    return jnp.transpose(outT, (2, 0, 1))


# speedup vs baseline: 7.4442x; 1.0006x over previous
import functools

import jax
import jax.numpy as jnp
from jax import lax
from jax.experimental import pallas as pl
from jax.experimental.pallas import tpu as pltpu
from jax.experimental.pallas import tpu_sc as plsc

ENTITY_VOCAB = 1000000
EMBED_DIM = 64
BATCH = 4096
HIST = 50

_INFO = plsc.get_sparse_core_info()
_NC = _INFO.num_cores
_NS = _INFO.num_subcores
_NW = _NC * _NS
_NBLK = ENTITY_VOCAB // 128
_NBUF_A = 6
_NBUF_B = 4
_BPW = BATCH // _NW


def _iota16():
    return lax.iota(jnp.int32, 16)


def _wid():
    return lax.axis_index("s") * _NC + lax.axis_index("c")


def _permute_rows(tile_ref, pair_ref, nrows):
    yvecs = [_iota16() + 16 * t for t in range(nrows // 16)]
    two_i = _iota16() * 2

    @plsc.parallel_loop(0, 128, step=1, unroll=4)
    def kbody(k):
        rowv = jnp.full((16,), lax.bitwise_and(k, 63), jnp.int32)
        colb = two_i + lax.shift_right_logical(k, 6)
        for t, yv in enumerate(yvecs):
            v = plsc.load_gather(tile_ref, [rowv, colb + 32 * t])
            plsc.store_scatter(pair_ref,
                               [yv, lax.bitwise_and(yv + k, 127)], v)


def _transpose_body(tabT, TT, tile_v, pair_v, tail_in, tail_out, sem_in,
                    sem_out):
    w = _wid()
    base = 244 * w + jnp.minimum(w, 5)
    cnt = jnp.where(w < 5, 245, 244)
    cnt = jnp.where(w == _NW - 1, cnt - 1, cnt)
    jend = base + cnt

    def g_in(j, b):
        return pltpu.make_async_copy(
            tabT.at[:, pl.ds(128 * j, 128)], tile_v.at[b], sem_in.at[b])

    def g_out(j, b):
        return pltpu.make_async_copy(
            pair_v.at[b], TT.at[pl.ds(64 * j, 64), :], sem_out.at[b])

    for b in range(_NBUF_A):
        g_in(base + b, b).start()

    def step(t, carry):
        for b in range(_NBUF_A):
            j = base + _NBUF_A * t + b

            @pl.when(j < jend)
            def _():
                g_in(j, b).wait()

                @pl.when(t > 0)
                def _():
                    g_out(j - _NBUF_A, b).wait()

                _permute_rows(tile_v.at[b], pair_v.at[b], 64)
                g_out(j, b).start()

                @pl.when(j + _NBUF_A < jend)
                def _():
                    g_in(j + _NBUF_A, b).start()

        return carry

    lax.fori_loop(0, (245 + _NBUF_A - 1) // _NBUF_A, step, 0)

    for b in range(_NBUF_A):
        jlast = jend - 1 - lax.rem(jend - 1 - base - b, _NBUF_A)
        g_out(jlast, b).wait()

    @pl.when(w == _NW - 1)
    def _():
        pltpu.sync_copy(tabT.at[:, pl.ds(128 * _NBLK, 64)], tail_in)
        _permute_rows(tail_in, tail_out, 32)
        pltpu.sync_copy(tail_out, TT.at[pl.ds(64 * _NBLK, 32), :])


@jax.jit
def _phase_a(tabT):
    mesh = plsc.VectorSubcoreMesh(core_axis_name="c", subcore_axis_name="s")
    fn = pl.kernel(
        _transpose_body,
        mesh=mesh,
        out_type=jax.ShapeDtypeStruct((ENTITY_VOCAB // 2, 128), jnp.float32),
        scratch_types=[
            pltpu.VMEM((_NBUF_A, 64, 128), jnp.float32),
            pltpu.VMEM((_NBUF_A, 64, 128), jnp.float32),
            pltpu.VMEM((64, 64), jnp.float32),
            pltpu.VMEM((32, 128), jnp.float32),
            pltpu.SemaphoreType.DMA((_NBUF_A,)),
            pltpu.SemaphoreType.DMA((_NBUF_A,)),
        ],
        compiler_params=pltpu.CompilerParams(needs_layout_passes=False),
    )
    return fn(tabT)


def _gather_body(TT, idxT, outT, idx_v, p_v, half_v, rows_v, otile_v,
                 sem_rows, sem_out):
    w = _wid()
    pltpu.sync_copy(idxT.at[:, pl.ds(128 * w, 128)], idx_v)

    @plsc.parallel_loop(0, HIST, step=1, unroll=2)
    def prep(t):
        for m in range(8):
            r = idx_v[t, pl.ds(16 * m, 16)]
            p = lax.shift_right_logical(r, 1)
            p_v[t, pl.ds(16 * m, 16)] = p
            half_v[t, pl.ds(16 * m, 16)] = (
                lax.shift_left(lax.bitwise_and(r, 1), 6)
                + lax.bitwise_and(p, 63))

    lanes = [_iota16() + 16 * m for m in range(8)]

    def g_rows(h, b):
        return pltpu.make_async_copy(TT.at[p_v.at[h]], rows_v.at[b],
                                     sem_rows.at[b])

    def g_out(h, b):
        return pltpu.make_async_copy(
            otile_v.at[b], outT.at[h].at[:, pl.ds(128 * w, 128)],
            sem_out.at[b])

    for b in range(_NBUF_B):
        g_rows(b, b).start()

    def step(t, carry):
        for b in range(_NBUF_B):
            h = _NBUF_B * t + b

            @pl.when(h < HIST)
            def _():
                g_rows(h, b).wait()

                @pl.when(t > 0)
                def _():
                    g_out(h - _NBUF_B, b).wait()

                cols = [half_v[h, pl.ds(16 * m, 16)] for m in range(8)]

                @plsc.parallel_loop(0, EMBED_DIM, step=1, unroll=8)
                def cloop(c):
                    for m in range(8):
                        col = lax.bitwise_and(cols[m] + c, 127)
                        v = plsc.load_gather(rows_v.at[b], [lanes[m], col])
                        otile_v[b, c, pl.ds(16 * m, 16)] = v
                g_out(h, b).start()

                @pl.when(h + _NBUF_B < HIST)
                def _():
                    g_rows(h + _NBUF_B, b).start()

        return carry

    lax.fori_loop(0, (HIST + _NBUF_B - 1) // _NBUF_B, step, 0)

    for b in range(_NBUF_B):
        hlast = HIST - 1 - lax.rem(jnp.int32(HIST - 1 - b), _NBUF_B)
        g_out(hlast, b).wait()


@jax.jit
def _phase_b(TT, idxT):
    mesh = plsc.VectorSubcoreMesh(core_axis_name="c", subcore_axis_name="s")
    fn = pl.kernel(
        _gather_body,
        mesh=mesh,
        out_type=jax.ShapeDtypeStruct((HIST, EMBED_DIM, BATCH), jnp.float32),
        scratch_types=[
            pltpu.VMEM((HIST, 128), jnp.int32),
            pltpu.VMEM((HIST, 128), jnp.int32),
            pltpu.VMEM((HIST, 128), jnp.int32),
            pltpu.VMEM((_NBUF_B, 128, 128), jnp.float32),
            pltpu.VMEM((_NBUF_B, EMBED_DIM, 128), jnp.float32),
            pltpu.SemaphoreType.DMA((_NBUF_B,)),
            pltpu.SemaphoreType.DMA((_NBUF_B,)),
        ],
        compiler_params=pltpu.CompilerParams(needs_layout_passes=False),
    )
    return fn(TT, idxT)


def kernel(entities, entity_table):
    tabT = entity_table.T
    TT = _phase_a(tabT)
    idxT = entities.T
    outT = _phase_b(TT, idxT)
    return jnp.transpose(outT, (2, 0, 1))
